# baseline (speedup 1.0000x reference)
import functools

import jax
import jax.numpy as jnp
from jax import lax
from jax.experimental import pallas as pl
from jax.experimental.pallas import tpu as pltpu
from jax.experimental.pallas import tpu_sc as plsc

BATCH = 16384
SEQ = 50
DIM = 32
N_ROWS = BATCH * SEQ
NC, NS = 2, 16
NW = NC * NS
ELEMS_W = BATCH // NW
CB = 32
N_CHUNKS = ELEMS_W // CB
ROWS_C = CB * SEQ
EL_F = DIM * SEQ


PAD_R = 52
PAD_F = PAD_R * DIM


def _gather_body(idx_hbm, table_hbm, out_hbm, idx_v, rows_v, sem):
    wid = lax.axis_index("s") * NC + lax.axis_index("c")

    def chunk(c, _):
        e0 = wid * ELEMS_W + c * CB
        pltpu.sync_copy(idx_hbm.at[pl.ds(e0, CB)], idx_v)
        copies = [
            pltpu.async_copy(
                table_hbm.at[idx_v.at[e]],
                rows_v.at[pl.ds(e * PAD_R, SEQ)],
                sem,
            )
            for e in range(CB)
        ]
        for cp in copies:
            cp.wait()
        r0 = pl.multiple_of(e0 * PAD_R, 8)
        pltpu.sync_copy(rows_v, out_hbm.at[pl.ds(r0, CB * PAD_R)])
        return 0

    lax.fori_loop(0, N_CHUNKS, chunk, 0)


_gather = functools.partial(
    pl.kernel,
    mesh=plsc.VectorSubcoreMesh(core_axis_name="c", subcore_axis_name="s"),
    out_type=jax.ShapeDtypeStruct((BATCH * PAD_R, DIM), jnp.float32),
    scratch_types=[
        pltpu.VMEM((CB, SEQ), jnp.int32),
        pltpu.VMEM((CB * PAD_R, DIM), jnp.float32),
        pltpu.SemaphoreType.DMA,
    ],
    compiler_params=pltpu.CompilerParams(
        use_tc_tiling_on_sc=False, needs_layout_passes=False
    ),
)(_gather_body)


_ZC = 32768
_ZG = (1000000 + _ZC - 1) // _ZC


def _zprep_body(x_ref, o_ref):
    x = x_ref[...]
    d_i = lax.broadcasted_iota(jnp.int32, (DIM, 128), 0)
    c_i = lax.broadcasted_iota(jnp.int32, (DIM, 128), 1)
    sel = (d_i == c_i % DIM).astype(jnp.float32)
    o1 = lax.dot_general(
        x, sel, (((0,), (0,)), ((), ())),
        preferred_element_type=jnp.float32,
    )
    o2 = jnp.reshape(o1, (_ZC // 4, 4, 128))
    lane = lax.broadcasted_iota(jnp.int32, (1, 128), 1)
    acc = jnp.zeros((_ZC // 4, 128), jnp.float32)
    for e in range(4):
        m_e = (lane // DIM == e).astype(jnp.float32)
        acc = acc + o2[:, e, :] * m_e
    o_ref[...] = acc


_zprep = pl.pallas_call(
    _zprep_body,
    grid=(_ZG,),
    in_specs=[pl.BlockSpec((DIM, _ZC), lambda k: (0, k))],
    out_specs=pl.BlockSpec((_ZC // 4, 128), lambda k: (k, 0)),
    out_shape=jax.ShapeDtypeStruct((250000, 128), jnp.float32),
)


_X2_ROWS = BATCH * PAD_R * DIM // 128
_BLK = 13 * 128


def _t2_body(x_ref, o_ref):
    y = jnp.reshape(x_ref[...], (128, 13, 128))
    eye = jnp.eye(128, dtype=jnp.float32)
    w = lax.dot_general(
        y, eye, (((0,), (0,)), ((), ())),
        preferred_element_type=jnp.float32,
    )
    o_ref[...] = jnp.reshape(w, (PAD_F, 128))[:EL_F]


_t2 = pl.pallas_call(
    _t2_body,
    grid=(BATCH // 128,),
    in_specs=[pl.BlockSpec((_BLK, 128), lambda k: (k, 0))],
    out_specs=pl.BlockSpec((EL_F, 128), lambda k: (0, k)),
    out_shape=jax.ShapeDtypeStruct((EL_F, BATCH), jnp.float32),
)


def kernel(inputs, z):
    w = _zprep(jnp.transpose(z))
    table = jnp.reshape(w, (1000000, DIM))
    g = _gather(inputs.astype(jnp.int32), table)
    x2 = jnp.reshape(g, (_X2_ROWS, 128))
    p2 = _t2(x2)
    return jnp.transpose(jnp.reshape(p2, (SEQ, DIM, BATCH)), (2, 1, 0))


# speedup vs baseline: 1.0976x; 1.0976x over previous
import functools

import jax
import jax.numpy as jnp
from jax import lax
from jax.experimental import pallas as pl
from jax.experimental.pallas import tpu as pltpu
from jax.experimental.pallas import tpu_sc as plsc

BATCH = 16384
SEQ = 50
DIM = 32
N_ROWS = BATCH * SEQ
NC, NS = 2, 16
NW = NC * NS
ELEMS_W = BATCH // NW
CB = 32
N_CHUNKS = ELEMS_W // CB
ROWS_C = CB * SEQ
EL_F = DIM * SEQ


PAD_R = 52
PAD_F = PAD_R * DIM


def _gather_body(idx_hbm, table_hbm, out_hbm, idx_v, rows_v, sem):
    wid = lax.axis_index("s") * NC + lax.axis_index("c")

    def chunk(c, _):
        e0 = wid * ELEMS_W + c * CB
        pltpu.sync_copy(idx_hbm.at[pl.ds(e0, CB)], idx_v)
        copies = [
            pltpu.async_copy(
                table_hbm.at[idx_v.at[e]],
                rows_v.at[pl.ds(e * PAD_R, SEQ)],
                sem,
            )
            for e in range(CB)
        ]
        for cp in copies:
            cp.wait()
        r0 = pl.multiple_of(e0 * PAD_R, 8)
        pltpu.sync_copy(rows_v, out_hbm.at[pl.ds(r0, CB * PAD_R)])
        return 0

    lax.fori_loop(0, N_CHUNKS, chunk, 0)


_gather = functools.partial(
    pl.kernel,
    mesh=plsc.VectorSubcoreMesh(core_axis_name="c", subcore_axis_name="s"),
    out_type=jax.ShapeDtypeStruct((BATCH * PAD_R, DIM), jnp.float32),
    scratch_types=[
        pltpu.VMEM((CB, SEQ), jnp.int32),
        pltpu.VMEM((CB * PAD_R, DIM), jnp.float32),
        pltpu.SemaphoreType.DMA,
    ],
    compiler_params=pltpu.CompilerParams(
        use_tc_tiling_on_sc=False, needs_layout_passes=False
    ),
)(_gather_body)


_ZC = 32768
_ZG = (1000000 + _ZC - 1) // _ZC


def _zprep_body(x_ref, o_ref):
    x = x_ref[...]
    eye = jnp.eye(DIM, dtype=jnp.float32)
    t = lax.dot_general(
        x, eye, (((0,), (0,)), ((), ())),
        preferred_element_type=jnp.float32,
    )
    t3 = jnp.reshape(t, (_ZC // 4, 4, DIM))
    d_i = lax.broadcasted_iota(jnp.int32, (DIM, 128), 0)
    c_i = lax.broadcasted_iota(jnp.int32, (DIM, 128), 1)
    acc = jnp.zeros((_ZC // 4, 128), jnp.float32)
    for e in range(4):
        sel_e = (c_i == e * DIM + d_i).astype(jnp.float32)
        acc = acc + lax.dot_general(
            t3[:, e, :], sel_e, (((1,), (0,)), ((), ())),
            preferred_element_type=jnp.float32,
        )
    o_ref[...] = acc


_zprep = pl.pallas_call(
    _zprep_body,
    grid=(_ZG,),
    in_specs=[pl.BlockSpec((DIM, _ZC), lambda k: (0, k))],
    out_specs=pl.BlockSpec((_ZC // 4, 128), lambda k: (k, 0)),
    out_shape=jax.ShapeDtypeStruct((250000, 128), jnp.float32),
)


_X2_ROWS = BATCH * PAD_R * DIM // 128
_BLK = 13 * 128


def _t2_body(x_ref, o_ref):
    y = jnp.reshape(x_ref[...], (128, 13, 128))
    eye = jnp.eye(128, dtype=jnp.float32)
    w = lax.dot_general(
        y, eye, (((0,), (0,)), ((), ())),
        preferred_element_type=jnp.float32,
    )
    o_ref[...] = jnp.reshape(w, (PAD_F, 128))[:EL_F]


_t2 = pl.pallas_call(
    _t2_body,
    grid=(BATCH // 128,),
    in_specs=[pl.BlockSpec((_BLK, 128), lambda k: (k, 0))],
    out_specs=pl.BlockSpec((EL_F, 128), lambda k: (0, k)),
    out_shape=jax.ShapeDtypeStruct((EL_F, BATCH), jnp.float32),
)


def kernel(inputs, z):
    w = _zprep(jnp.transpose(z))
    table = jnp.reshape(w, (1000000, DIM))
    g = _gather(inputs.astype(jnp.int32), table)
    x2 = jnp.reshape(g, (_X2_ROWS, 128))
    p2 = _t2(x2)
    return jnp.transpose(jnp.reshape(p2, (SEQ, DIM, BATCH)), (2, 1, 0))
